# baseline (speedup 1.0000x reference)
import functools
import math

import jax
import jax.numpy as jnp
from jax import lax
from jax.experimental import pallas as pl
from jax.experimental.pallas import tpu as pltpu
from jax.experimental.pallas import tpu_sc as plsc

NUM_EMB = 1000000
DIM = 64
SCALE = math.sqrt(DIM)

_info = plsc.get_sparse_core_info()
NC, NS, L = _info.num_cores, _info.num_subcores, _info.num_lanes
NW = NC * NS

IB = 128
D8 = DIM // 8
BLK = 8 * IB
SG = 4
GR = SG * IB
RU = 16


def _make_kernel(S, NI):
    assert NI == NW and S % SG == 0 and IB % RU == 0
    KMAX = S // SG
    mesh = plsc.VectorSubcoreMesh(core_axis_name="c", subcore_axis_name="s")

    @functools.partial(
        pl.kernel,
        mesh=mesh,
        out_type=jax.ShapeDtypeStruct((S, D8, NI, BLK), jnp.float32),
        scratch_types=[
            pltpu.VMEM((2, GR), jnp.int32),
            pltpu.VMEM((2, GR, DIM), jnp.float32),
            pltpu.VMEM((2, D8, BLK), jnp.float32),
            pltpu.SemaphoreType.DMA((2,)),
            pltpu.SemaphoreType.DMA((2,)),
            pltpu.SemaphoreType.DMA((2,)),
        ],
        compiler_params=pltpu.CompilerParams(
            use_tc_tiling_on_sc=False, needs_layout_passes=False
        ),
    )
    def k(x_hbm, table_hbm, out_hbm, idx_v, rows_v, obuf_v, isem, gsem, ssem):
        w = lax.axis_index("s") * NC + lax.axis_index("c")
        col = w * IB

        def load_idx(kk, b):
            for sl in range(SG):
                pltpu.make_async_copy(
                    x_hbm.at[kk * SG + sl, pl.ds(col, IB)],
                    idx_v.at[b, pl.ds(sl * IB, IB)],
                    isem.at[b],
                ).start()

        def wait_idx(b):
            for _ in range(SG):
                pltpu.make_async_copy(
                    x_hbm.at[0, pl.ds(0, IB)],
                    idx_v.at[0, pl.ds(0, IB)],
                    isem.at[b],
                ).wait()

        def start_gather(b):
            pltpu.make_async_copy(
                table_hbm.at[idx_v.at[b]], rows_v.at[b], gsem.at[b]
            ).start()

        load_idx(0, 0)
        load_idx(1, 1)
        wait_idx(0)
        start_gather(0)

        lanes = lax.iota(jnp.int32, L)

        def item(kk, carry):
            b = kk % 2
            bn = (kk + 1) % 2

            @pl.when(kk + 1 < KMAX)
            def _():
                wait_idx(bn)
                start_gather(bn)

            pltpu.make_async_copy(
                table_hbm.at[idx_v.at[0]], rows_v.at[b], gsem.at[b]
            ).wait()

            @pl.when(kk + 2 < KMAX)
            def _():
                load_idx(kk + 2, b)

            rows = rows_v.at[b]
            for sl in range(SG):
                p = sl % 2
                obuf = obuf_v.at[p]

                def trans(rb, c2, sl=sl, obuf=obuf):
                    rloc = rb * L + lanes
                    rowvec = sl * IB + rloc
                    for d0 in range(L):
                        dd = (d0 + lanes) & 15
                        maj0 = dd >> 3
                        min0 = ((dd & 7) << 7) + rloc
                        for d16 in range(DIM // L):
                            colvec = dd + d16 * L
                            v = plsc.load_gather(rows, [rowvec, colvec])
                            plsc.store_scatter(
                                obuf, [maj0 + d16 * 2, min0], v * SCALE
                            )
                    return c2

                lax.fori_loop(0, IB // L, trans, 0, unroll=False)

                @pl.when((kk > 0) | (sl >= 2))
                def _():
                    pltpu.make_async_copy(
                        obuf_v.at[p], out_hbm.at[0, :, 0], ssem.at[p]
                    ).wait()

                pltpu.make_async_copy(
                    obuf, out_hbm.at[kk * SG + sl, :, w], ssem.at[p]
                ).start()
            return carry

        lax.fori_loop(0, KMAX, item, 0)
        for p in range(2):
            pltpu.make_async_copy(
                obuf_v.at[p], out_hbm.at[0, :, 0], ssem.at[p]
            ).wait()

    return k


@jax.jit
def kernel(x, table):
    NB, SEQ = x.shape
    xt = jnp.asarray(x, jnp.int32).T
    out4 = _make_kernel(SEQ, NB // IB)(xt, table)
    out5 = out4.reshape(SEQ, D8, NB // IB, 8, IB)
    out = out5.transpose(2, 4, 0, 1, 3).reshape(NB, SEQ, DIM)
    return out
